# Initial kernel scaffold; baseline (speedup 1.0000x reference)
import jax
import jax.numpy as jnp
from jax.experimental import pallas as pl


def kernel(node_types, edge_index, embed, W_msg, Wr, Ur, Wz, Uz, Wn, Un, ln_g, ln_b, dec_W1, dec_b1, dec_W2, dec_b2):
    raise NotImplementedError("write your pallas kernel here")


# speedup vs baseline: 7.3743x; 7.3743x over previous
import functools

import jax
import jax.numpy as jnp
from jax import lax
from jax.experimental import pallas as pl
from jax.experimental.pallas import tpu as pltpu
from jax.experimental.pallas import tpu_sc as plsc

N = 100000
E = 1600000
HID = 32
MAXCH = 64

NHALF = N // 2
CHUNK = 128
QS = 2
SUPER = CHUNK * QS
NSUPER = E // SUPER
NSUB = 16
ZROWS = 80
NZCH = NHALF // ZROWS
DUMMY = NHALF
HPAD = NHALF + 8

BLK = 2000


def _segment_sum_sc(y, src, dst):
    mesh = plsc.VectorSubcoreMesh(core_axis_name="c", subcore_axis_name="s")

    @functools.partial(
        pl.kernel,
        mesh=mesh,
        compiler_params=pltpu.CompilerParams(use_tc_tiling_on_sc=False),
        out_type=jax.ShapeDtypeStruct((N, HID), jnp.float32),
        scratch_types=[
            pltpu.VMEM((2, SUPER), jnp.int32),
            pltpu.VMEM((2, SUPER), jnp.int32),
            pltpu.VMEM((2 * QS, CHUNK), jnp.int32),
            pltpu.VMEM((2, SUPER, HID), jnp.float32),
            pltpu.VMEM((ZROWS, HID), jnp.float32),
            pltpu.VMEM_SHARED((HPAD, HID), jnp.float32),
            pltpu.SemaphoreType.DMA,
            pltpu.SemaphoreType.DMA,
            pltpu.SemaphoreType.DMA,
        ],
    )
    def k(y_hbm, src_hbm, dst_hbm, out_hbm, src_v, dst_raw, dst_v, rows_v,
          zbuf, h_sh, sem_i, sem_g, sem_s):
        c = lax.axis_index("c")
        s = lax.axis_index("s")
        base = c * NHALF

        zeros = jnp.zeros((16,), jnp.float32)
        for r in range(ZROWS):
            for q in range(HID // 16):
                zbuf[r, pl.ds(q * 16, 16)] = zeros

        def zbody(j, carry):
            r0 = pl.multiple_of((s + j * NSUB) * ZROWS, 8)
            pltpu.sync_copy(zbuf, h_sh.at[pl.ds(r0, ZROWS)])
            return carry

        nz_mine = (NZCH - s + NSUB - 1) // NSUB
        lax.fori_loop(0, nz_mine, zbody, 0)

        @pl.when(s == 0)
        def _():
            pltpu.sync_copy(zbuf.at[pl.ds(0, HPAD - NHALF)],
                            h_sh.at[pl.ds(NHALF, HPAD - NHALF)])

        plsc.subcore_barrier()

        n_mine = (NSUPER - s + NSUB - 1) // NSUB

        def _drain_scatter(p, q):
            pltpu.make_async_copy(
                rows_v.at[p, pl.ds(q * CHUNK, CHUNK)],
                h_sh.at[dst_v.at[p * QS + q]], sem_s).wait()

        def _issue_idx(j, p):
            off = pl.multiple_of((s + j * NSUB) * SUPER, 8)
            pltpu.async_copy(src_hbm.at[pl.ds(off, SUPER)], src_v.at[p], sem_i)
            pltpu.async_copy(dst_hbm.at[pl.ds(off, SUPER)], dst_raw.at[p],
                             sem_i)

        def _wait_idx(j, p):
            off = pl.multiple_of((s + j * NSUB) * SUPER, 8)
            pltpu.make_async_copy(src_hbm.at[pl.ds(off, SUPER)], src_v.at[p],
                                  sem_i).wait()
            pltpu.make_async_copy(dst_hbm.at[pl.ds(off, SUPER)],
                                  dst_raw.at[p], sem_i).wait()

        def _clamp_dst(p):
            for q in range(QS):
                for r in range(CHUNK // 16):
                    d = dst_raw[p, pl.ds(q * CHUNK + r * 16, 16)]
                    inr = (d >= base) & (d < base + NHALF)
                    dst_v[p * QS + q, pl.ds(r * 16, 16)] = (
                        jnp.where(inr, d - base, DUMMY))

        def _fire_gather(p):
            for q in range(QS):
                pltpu.async_copy(
                    y_hbm.at[src_v.at[p, pl.ds(q * CHUNK, CHUNK)]],
                    rows_v.at[p, pl.ds(q * CHUNK, CHUNK)], sem_g)

        def _wait_gather(p):
            for q in range(QS):
                pltpu.make_async_copy(
                    y_hbm.at[src_v.at[p, pl.ds(q * CHUNK, CHUNK)]],
                    rows_v.at[p, pl.ds(q * CHUNK, CHUNK)], sem_g).wait()

        def _fire_scatter(p):
            for q in range(QS):
                pltpu.async_copy(
                    rows_v.at[p, pl.ds(q * CHUNK, CHUNK)],
                    h_sh.at[dst_v.at[p * QS + q]], sem_s, add=True)

        _issue_idx(0, 0)
        _wait_idx(0, 0)
        _clamp_dst(0)
        _fire_gather(0)

        @pl.when(n_mine > 1)
        def _():
            _issue_idx(1, 1)

        def body(j, carry):
            p = j % 2

            @pl.when(j >= 1)
            def _():
                for q in range(QS):
                    _drain_scatter(1 - p, q)

            @pl.when(j + 1 < n_mine)
            def _():
                _wait_idx(j + 1, 1 - p)
                _clamp_dst(1 - p)
                _fire_gather(1 - p)

            _wait_gather(p)

            @pl.when(j + 2 < n_mine)
            def _():
                _issue_idx(j + 2, p)

            _fire_scatter(p)
            return carry

        lax.fori_loop(0, n_mine, body, 0)

        for q in range(QS):
            _drain_scatter((n_mine - 1) % 2, q)

        plsc.subcore_barrier()

        def obody(j, carry):
            r0 = pl.multiple_of((s + j * NSUB) * ZROWS, 8)
            pltpu.sync_copy(h_sh.at[pl.ds(r0, ZROWS)],
                            out_hbm.at[pl.ds(base + r0, ZROWS)])
            return carry

        lax.fori_loop(0, nz_mine, obody, 0)

    return k(y, src, dst)


def _dot(a, b):
    return jnp.dot(a, b, preferred_element_type=jnp.float32)


def _gru(x, h, wr, ur, wz, uz, wn, un):
    r = jax.nn.sigmoid(_dot(h, wr) + _dot(x, ur))
    z = jax.nn.sigmoid(_dot(h, wz) + _dot(x, uz))
    n = jnp.tanh(_dot(h, wn) + r * _dot(x, un))
    return (1.0 - z) * n + z * x


def _row_spec():
    return pl.BlockSpec((BLK, HID), lambda i: (i, 0))


def _full_spec(shape):
    return pl.BlockSpec(shape, lambda i: (0,) * len(shape))


def _stage_embed(node_types2d, embed_pad, w_msg):
    def body(nt_ref, emb_ref, wm_ref, x_ref, y_ref):
        nt = nt_ref[...]
        oh = (nt == lax.broadcasted_iota(jnp.int32, (BLK, 16), 1)).astype(jnp.float32)
        x = _dot(oh, emb_ref[...])
        x_ref[...] = x
        y_ref[...] = jnp.maximum(_dot(x, wm_ref[...]), 0.0)

    return pl.pallas_call(
        body,
        grid=(N // BLK,),
        in_specs=[pl.BlockSpec((BLK, 1), lambda i: (i, 0)),
                  _full_spec((16, HID)),
                  _full_spec((HID, HID))],
        out_specs=[_row_spec(), _row_spec()],
        out_shape=[jax.ShapeDtypeStruct((N, HID), jnp.float32),
                   jax.ShapeDtypeStruct((N, HID), jnp.float32)],
    )(node_types2d, embed_pad, w_msg)


def _stage_gru_msg(x, h, wr, ur, wz, uz, wn, un, w_msg):
    def body(x_ref, h_ref, wr_r, ur_r, wz_r, uz_r, wn_r, un_r, wm_r,
             x1_ref, y1_ref):
        xn = _gru(x_ref[...], h_ref[...], wr_r[...], ur_r[...], wz_r[...],
                  uz_r[...], wn_r[...], un_r[...])
        x1_ref[...] = xn
        y1_ref[...] = jnp.maximum(_dot(xn, wm_r[...]), 0.0)

    w_spec = _full_spec((HID, HID))
    return pl.pallas_call(
        body,
        grid=(N // BLK,),
        in_specs=[_row_spec(), _row_spec()] + [w_spec] * 7,
        out_specs=[_row_spec(), _row_spec()],
        out_shape=[jax.ShapeDtypeStruct((N, HID), jnp.float32),
                   jax.ShapeDtypeStruct((N, HID), jnp.float32)],
    )(x, h, wr, ur, wz, uz, wn, un, w_msg)


def _stage_final(x, h, wr, ur, wz, uz, wn, un, ln_g, ln_b, w1, b1, w2, b2):
    def body(x_ref, h_ref, wr_r, ur_r, wz_r, uz_r, wn_r, un_r,
             g_r, bb_r, w1_r, b1_r, w2_r, b2_r, out_ref):
        xn = _gru(x_ref[...], h_ref[...], wr_r[...], ur_r[...], wz_r[...],
                  uz_r[...], wn_r[...], un_r[...])
        mu = jnp.mean(xn, axis=-1, keepdims=True)
        var = jnp.mean((xn - mu) ** 2, axis=-1, keepdims=True)
        xs = (xn - mu) * jax.lax.rsqrt(var + 1e-5) * g_r[...] + bb_r[...]
        d = jnp.maximum(_dot(xs, w1_r[...]) + b1_r[...], 0.0)
        out_ref[...] = _dot(d, w2_r[...]) + b2_r[...]

    return pl.pallas_call(
        body,
        grid=(N // BLK,),
        in_specs=[_row_spec(), _row_spec()]
        + [_full_spec((HID, HID))] * 6
        + [_full_spec((1, HID)), _full_spec((1, HID)),
           _full_spec((HID, 2 * HID)), _full_spec((1, 2 * HID)),
           _full_spec((2 * HID, 2 * MAXCH)), _full_spec((1, 2 * MAXCH))],
        out_specs=[pl.BlockSpec((BLK, 2 * MAXCH), lambda i: (i, 0))],
        out_shape=[jax.ShapeDtypeStruct((N, 2 * MAXCH), jnp.float32)],
    )(x, h, wr, ur, wz, uz, wn, un, ln_g, ln_b, w1, b1, w2, b2)[0]


def kernel(node_types, edge_index, embed, W_msg, Wr, Ur, Wz, Uz, Wn, Un,
           ln_g, ln_b, dec_W1, dec_b1, dec_W2, dec_b2):
    src = edge_index[0]
    dst = edge_index[1]
    embed_pad = jnp.concatenate(
        [embed, jnp.zeros((16 - embed.shape[0], HID), jnp.float32)], axis=0)

    x0, y0 = _stage_embed(node_types.reshape(N, 1), embed_pad, W_msg)
    h1 = _segment_sum_sc(y0, src, dst)
    x1, y1 = _stage_gru_msg(x0, h1, Wr, Ur, Wz, Uz, Wn, Un, W_msg)
    h2 = _segment_sum_sc(y1, dst, src)
    out = _stage_final(x1, h2, Wr, Ur, Wz, Uz, Wn, Un,
                       ln_g.reshape(1, HID), ln_b.reshape(1, HID),
                       dec_W1, dec_b1.reshape(1, 2 * HID),
                       dec_W2, dec_b2.reshape(1, 2 * MAXCH))
    return out.reshape(N, 2, MAXCH)
